# baseline (speedup 1.0000x reference)
import jax
import jax.numpy as jnp
from jax import lax
from jax.experimental import pallas as pl
from jax.experimental.pallas import tpu as pltpu
from jax.experimental.pallas import tpu_sc as plsc

_ROWS = 9216
_K = 1024
_D = 64
_TILE = 512
_KC = 128
_NSPLIT = 2
_HROWS = _ROWS // _NSPLIT
_NC, _NS = 2, 16
_NW = _NC * _NS
_BPW = _HROWS // _NW
_CHUNK = 72
_NCHUNK = _BPW // _CHUNK


def _argmin_body(x_ref, em2_ref, e2_ref, rowsq_ref, idx_ref):
    x = x_ref[...]
    rowsq = rowsq_ref[...]
    run_min = None
    for c in range(_K // _KC):
        sl = pl.ds(c * _KC, _KC)
        s = lax.dot_general(em2_ref[sl, :], x, (((1,), (1,)), ((), ())),
                            preferred_element_type=jnp.float32)
        d2 = (rowsq + s) + e2_ref[sl, :]
        row = lax.broadcasted_iota(jnp.int32, d2.shape, 0) + c * _KC
        if run_min is None:
            run_min, run_idx = d2, row
        else:
            better = d2 < run_min
            run_min = jnp.where(better, d2, run_min)
            run_idx = jnp.where(better, row, run_idx)
    m = jnp.min(run_min, axis=0, keepdims=True)
    idx_ref[...] = jnp.min(jnp.where(run_min == m, run_idx, _K), axis=0)


def _nearest_indices(flat, em2, e2, rowsq, half):
    base = half * (_HROWS // _TILE)
    return pl.pallas_call(
        _argmin_body,
        grid=(_HROWS // _TILE,),
        in_specs=[
            pl.BlockSpec((_TILE, _D), lambda i: (base + i, 0)),
            pl.BlockSpec((_K, _D), lambda i: (0, 0)),
            pl.BlockSpec((_K, 1), lambda i: (0, 0)),
            pl.BlockSpec((1, _TILE), lambda i: (0, base + i)),
        ],
        out_specs=pl.BlockSpec((_TILE,), lambda i: (i,)),
        out_shape=jax.ShapeDtypeStruct((_HROWS,), jnp.int32),
    )(flat, em2, e2, rowsq)


def _gather_body(table_hbm, idx_hbm, out_hbm, idx0, idx1, rows_v, sem):
    wid = lax.axis_index("s") * _NC + lax.axis_index("c")
    base = wid * _BPW
    bufs = (idx0, idx1)
    for c in range(_NCHUNK):
        pltpu.sync_copy(idx_hbm.at[pl.ds(base + c * _CHUNK, _CHUNK)], bufs[c])
    copies = [
        pltpu.async_copy(
            table_hbm.at[bufs[c]],
            rows_v.at[pl.ds(c * _CHUNK, _CHUNK)],
            sem,
        )
        for c in range(_NCHUNK)
    ]
    for cp in copies:
        cp.wait()
    pltpu.sync_copy(rows_v, out_hbm.at[pl.ds(base, _BPW)])


def _gather_rows(embedding, idx):
    return pl.kernel(
        _gather_body,
        out_type=jax.ShapeDtypeStruct((_HROWS, _D), jnp.float32),
        mesh=plsc.VectorSubcoreMesh(core_axis_name="c", subcore_axis_name="s"),
        compiler_params=pltpu.CompilerParams(use_tc_tiling_on_sc=False),
        scratch_types=[
            pltpu.VMEM((_CHUNK,), jnp.int32),
            pltpu.VMEM((_CHUNK,), jnp.int32),
            pltpu.VMEM((_BPW, _D), jnp.float32),
            pltpu.SemaphoreType.DMA,
        ],
    )(embedding, idx)


def kernel(encoded, embedding):
    bsz, T, dims = encoded.shape
    flat = encoded.reshape(bsz * T, dims)
    em2 = embedding * -2.0
    e2 = jnp.sum(embedding * embedding, axis=1)[:, None]
    rowsq = jnp.sum(flat * flat, axis=1)[None, :]
    halves = []
    for h in range(_NSPLIT):
        idx_h = _nearest_indices(flat, em2, e2, rowsq, h)
        halves.append(_gather_rows(embedding, idx_h))
    quantized = jnp.concatenate(halves, axis=0)
    return quantized.reshape(bsz, T, dims)


# speedup vs baseline: 1.0723x; 1.0723x over previous
import jax
import jax.numpy as jnp
from jax import lax
from jax.experimental import pallas as pl
from jax.experimental.pallas import tpu as pltpu
from jax.experimental.pallas import tpu_sc as plsc

_B, _T = 16, 576
_ROWS = _B * _T
_K = 1024
_D = 64
_TILE = 512
_KC = 128
_NC, _NS = 2, 16
_NW = _NC * _NS
_BPW = _ROWS // _NW
_CHUNK = 96
_NCHUNK = _BPW // _CHUNK


def _argmin_body(x_ref, em2_ref, e2_ref, rowsq_ref, idx_ref):
    x = x_ref[...]
    rowsq = rowsq_ref[...].reshape(1, _TILE)
    run_min = None
    for c in range(_K // _KC):
        sl = pl.ds(c * _KC, _KC)
        s = lax.dot_general(em2_ref[sl, :], x, (((1,), (1,)), ((), ())),
                            preferred_element_type=jnp.float32)
        d2 = (rowsq + s) + e2_ref[sl, :]
        row = lax.broadcasted_iota(jnp.int32, d2.shape, 0) + c * _KC
        if run_min is None:
            run_min, run_idx = d2, row
        else:
            better = d2 < run_min
            run_min = jnp.where(better, d2, run_min)
            run_idx = jnp.where(better, row, run_idx)
    m = jnp.min(run_min, axis=0, keepdims=True)
    idx_ref[...] = jnp.min(jnp.where(run_min == m, run_idx, _K), axis=0)


def _nearest_indices(flat, em2, e2, rowsq):
    return pl.pallas_call(
        _argmin_body,
        grid=(_ROWS // _TILE,),
        in_specs=[
            pl.BlockSpec((_TILE, _D), lambda i: (i, 0)),
            pl.BlockSpec((_K, _D), lambda i: (0, 0)),
            pl.BlockSpec((_K, 1), lambda i: (0, 0)),
            pl.BlockSpec((_TILE,), lambda i: (i,)),
        ],
        out_specs=pl.BlockSpec((_TILE,), lambda i: (i,)),
        out_shape=jax.ShapeDtypeStruct((_ROWS,), jnp.int32),
    )(flat, em2, e2, rowsq)


def _gather_body(table_hbm, idx_hbm, out_hbm, idx_v, rows_v, sem):
    wid = lax.axis_index("s") * _NC + lax.axis_index("c")
    base = wid * _BPW
    pltpu.sync_copy(idx_hbm.at[pl.ds(base, _BPW)], idx_v)
    copies = [
        pltpu.async_copy(
            table_hbm.at[idx_v.at[pl.ds(c * _CHUNK, _CHUNK)]],
            rows_v.at[pl.ds(c * _CHUNK, _CHUNK)],
            sem,
        )
        for c in range(_NCHUNK)
    ]
    for cp in copies:
        cp.wait()
    b = wid // 2
    h = wid % 2
    pltpu.sync_copy(rows_v, out_hbm.at[b, pl.ds(h * _BPW, _BPW)])


def _gather_rows(embedding, idx):
    return pl.kernel(
        _gather_body,
        out_type=jax.ShapeDtypeStruct((_B, _T, _D), jnp.float32),
        mesh=plsc.VectorSubcoreMesh(core_axis_name="c", subcore_axis_name="s"),
        compiler_params=pltpu.CompilerParams(use_tc_tiling_on_sc=False),
        scratch_types=[
            pltpu.VMEM((_BPW,), jnp.int32),
            pltpu.VMEM((_BPW, _D), jnp.float32),
            pltpu.SemaphoreType.DMA,
        ],
    )(embedding, idx)


def kernel(encoded, embedding):
    bsz, T, dims = encoded.shape
    flat = encoded.reshape(bsz * T, dims)
    em2 = embedding * -2.0
    e2 = jnp.sum(embedding * embedding, axis=1)[:, None]
    rowsq = jnp.sum(flat * flat, axis=1)
    idx = _nearest_indices(flat, em2, e2, rowsq)
    return _gather_rows(embedding, idx)


# speedup vs baseline: 1.0989x; 1.0248x over previous
import jax
import jax.numpy as jnp
from jax import lax
from jax.experimental import pallas as pl
from jax.experimental.pallas import tpu as pltpu
from jax.experimental.pallas import tpu_sc as plsc

_B, _T = 16, 576
_ROWS = _B * _T
_K = 1024
_D = 64
_TILE = 512
_KC = 128
_NC, _NS = 2, 16
_NW = _NC * _NS
_BPW = _ROWS // _NW
_CHUNK = 96
_NCHUNK = _BPW // _CHUNK


def _argmin_body(x_ref, em2_ref, e2_ref, idx_ref):
    x = x_ref[...]
    xsq = x * x
    rowsq = lax.dot_general(jnp.ones((1, _D), jnp.float32), xsq,
                            (((1,), (1,)), ((), ())),
                            preferred_element_type=jnp.float32)
    run_min = None
    for c in range(_K // _KC):
        sl = pl.ds(c * _KC, _KC)
        s = lax.dot_general(em2_ref[sl, :], x, (((1,), (1,)), ((), ())),
                            preferred_element_type=jnp.float32)
        d2 = (rowsq + s) + e2_ref[sl, :]
        row = lax.broadcasted_iota(jnp.int32, d2.shape, 0) + c * _KC
        if run_min is None:
            run_min, run_idx = d2, row
        else:
            better = d2 < run_min
            run_min = jnp.where(better, d2, run_min)
            run_idx = jnp.where(better, row, run_idx)
    m = jnp.min(run_min, axis=0, keepdims=True)
    idx_ref[...] = jnp.min(jnp.where(run_min == m, run_idx, _K), axis=0)


def _nearest_indices(flat, em2, e2):
    return pl.pallas_call(
        _argmin_body,
        grid=(_ROWS // _TILE,),
        in_specs=[
            pl.BlockSpec((_TILE, _D), lambda i: (i, 0)),
            pl.BlockSpec((_K, _D), lambda i: (0, 0)),
            pl.BlockSpec((_K, 1), lambda i: (0, 0)),
        ],
        out_specs=pl.BlockSpec((_TILE,), lambda i: (i,)),
        out_shape=jax.ShapeDtypeStruct((_ROWS,), jnp.int32),
    )(flat, em2, e2)


def _gather_body(table_hbm, idx_hbm, out_hbm, idx_v, rows_v, sem):
    wid = lax.axis_index("s") * _NC + lax.axis_index("c")
    base = wid * _BPW
    pltpu.sync_copy(idx_hbm.at[pl.ds(base, _BPW)], idx_v)
    copies = [
        pltpu.async_copy(
            table_hbm.at[idx_v.at[pl.ds(c * _CHUNK, _CHUNK)]],
            rows_v.at[pl.ds(c * _CHUNK, _CHUNK)],
            sem,
        )
        for c in range(_NCHUNK)
    ]
    for cp in copies:
        cp.wait()
    b = wid // 2
    h = wid % 2
    pltpu.sync_copy(rows_v, out_hbm.at[b, pl.ds(h * _BPW, _BPW)])


def _gather_rows(embedding, idx):
    return pl.kernel(
        _gather_body,
        out_type=jax.ShapeDtypeStruct((_B, _T, _D), jnp.float32),
        mesh=plsc.VectorSubcoreMesh(core_axis_name="c", subcore_axis_name="s"),
        compiler_params=pltpu.CompilerParams(use_tc_tiling_on_sc=False),
        scratch_types=[
            pltpu.VMEM((_BPW,), jnp.int32),
            pltpu.VMEM((_BPW, _D), jnp.float32),
            pltpu.SemaphoreType.DMA,
        ],
    )(embedding, idx)


def kernel(encoded, embedding):
    bsz, T, dims = encoded.shape
    flat = encoded.reshape(bsz * T, dims)
    em2 = embedding * -2.0
    e2 = jnp.sum(embedding * embedding, axis=1)[:, None]
    idx = _nearest_indices(flat, em2, e2)
    return _gather_rows(embedding, idx)
